# Initial kernel scaffold; baseline (speedup 1.0000x reference)
import jax
import jax.numpy as jnp
from jax.experimental import pallas as pl


def kernel(pred, target, object_num):
    raise NotImplementedError("write your pallas kernel here")


# speedup vs baseline: 1.0744x; 1.0744x over previous
import functools

import jax
import jax.numpy as jnp
from jax import lax
from jax.experimental import pallas as pl
from jax.experimental.pallas import tpu as pltpu
from jax.experimental.pallas import tpu_sc as plsc

_B = 16
_Q = 4096
_L = 16
_NITER = _Q // _L

_LOG1P_COEFS = (
    2.2159764878626476e-07,
    0.9999702432977379,
    -0.49933394898196387,
    0.32751171370195564,
    -0.22396689943001968,
    0.13198966240017918,
    -0.05326747773424277,
    0.01024382863142621,
)


def _ce_body(pred_hbm, tgt_hbm, objn_hbm, out_hbm,
             row_v, tgt_v, objn_v, stage_v, allrows_v, shared):
    s = lax.axis_index("s")
    pltpu.sync_copy(pred_hbm.at[s], row_v)
    pltpu.sync_copy(tgt_hbm.at[s], tgt_v)
    pltpu.sync_copy(objn_hbm, objn_v)

    iota = lax.broadcasted_iota(jnp.int32, (_L,), 0)
    sful = jnp.full((_L,), s, jnp.int32)
    my_numb = plsc.load_gather(objn_v, [sful])
    inv = (1.0 / _B) / my_numb.astype(jnp.float32)
    two_iota = iota * 2

    def step(i, acc):
        tok = iota + i * _L
        idx0 = two_iota + i * (2 * _L)
        g0 = plsc.load_gather(row_v, [idx0])
        g1 = plsc.load_gather(row_v, [idx0 + 1])
        t = plsc.load_gather(tgt_v, [tok])
        m = jnp.maximum(g0, g1)
        e = jnp.exp(jnp.minimum(g0, g1) - m)
        lp = jnp.full((_L,), _LOG1P_COEFS[-1], jnp.float32)
        for c in _LOG1P_COEFS[-2::-1]:
            lp = lp * e + c
        pt = jnp.where(t == 0, g0, g1)
        ce = m + lp - pt
        return acc + jnp.where(tok < my_numb, ce, 0.0)

    acc = lax.fori_loop(0, _NITER, step, jnp.zeros((_L,), jnp.float32))
    stage_v[...] = acc * inv
    pltpu.sync_copy(stage_v, shared.at[pl.ds(s * _L, _L)])
    plsc.subcore_barrier()

    @pl.when(s == 0)
    def _():
        pltpu.sync_copy(shared, allrows_v)
        tot = jnp.zeros((_L,), jnp.float32)
        for ss in range(_B):
            tot = tot + allrows_v[pl.ds(ss * _L, _L)]
        stage_v[...] = jnp.full((_L,), jnp.sum(tot), jnp.float32)
        pltpu.sync_copy(stage_v, out_hbm)


_sc_celoss = functools.partial(
    pl.kernel,
    out_type=jax.ShapeDtypeStruct((_L,), jnp.float32),
    mesh=plsc.VectorSubcoreMesh(
        core_axis_name="c", subcore_axis_name="s", num_cores=1),
    compiler_params=pltpu.CompilerParams(needs_layout_passes=False),
    scratch_types=[
        pltpu.VMEM((2 * _Q,), jnp.float32),
        pltpu.VMEM((_Q,), jnp.int32),
        pltpu.VMEM((_B,), jnp.int32),
        pltpu.VMEM((_L,), jnp.float32),
        pltpu.VMEM((_B * _L,), jnp.float32),
        pltpu.VMEM_SHARED((_B * _L,), jnp.float32),
    ],
)(_ce_body)


def kernel(pred, target, object_num):
    pred2 = pred.reshape(_B, 2 * _Q)
    out = _sc_celoss(pred2, target.astype(jnp.int32),
                     object_num.astype(jnp.int32))
    return out[0]
